# baseline (speedup 1.0000x reference)
import functools

import jax
import jax.numpy as jnp
from jax import lax
from jax.experimental import pallas as pl
from jax.experimental.pallas import tpu as pltpu
from jax.experimental.pallas import tpu_sc as plsc

N_OCT = 8
N_CHR = 12
D_HALF = 128
D_PROJ = 512
V_PAD = 128
SCALE = float(D_PROJ ** 0.5)

NC = 2
NS = 16
NW = NC * NS
L = 16

B_TOTAL = 1024 * 200
B_PER_W = B_TOTAL // NW
CHUNK = 32
N_CHUNKS = B_PER_W // CHUNK
ROW_VECS = D_PROJ // L


def _build_table_body(oct_lut_ref, chr_lut_ref, oct_tab_ref, chr_tab_ref,
                      w_ref, f_ref):
    oct_ids = oct_lut_ref[...]
    chr_ids = chr_lut_ref[...]
    iota16 = lax.broadcasted_iota(jnp.int32, (V_PAD, 16), 1)
    oh_oct = (oct_ids == iota16).astype(jnp.float32)
    oh_chr = (chr_ids == iota16).astype(jnp.float32)
    emb_oct = jnp.dot(oh_oct, oct_tab_ref[...],
                      preferred_element_type=jnp.float32)
    emb_chr = jnp.dot(oh_chr, chr_tab_ref[...],
                      preferred_element_type=jnp.float32)
    emb = jnp.concatenate([emb_oct, emb_chr], axis=1)
    f_ref[...] = jnp.dot(emb, w_ref[...],
                         preferred_element_type=jnp.float32) * SCALE


def _build_table(oct_lut, chr_lut, octave_table, chroma_table, w_proj):
    oct_lut_p = jnp.concatenate(
        [oct_lut, jnp.full((V_PAD - oct_lut.shape[0],), N_OCT, jnp.int32)]
    ).reshape(V_PAD, 1)
    chr_lut_p = jnp.concatenate(
        [chr_lut, jnp.full((V_PAD - chr_lut.shape[0],), N_CHR, jnp.int32)]
    ).reshape(V_PAD, 1)
    oct_tab_p = jnp.zeros((16, D_HALF), jnp.float32).at[:N_OCT + 1].set(octave_table)
    chr_tab_p = jnp.zeros((16, D_HALF), jnp.float32).at[:N_CHR + 1].set(chroma_table)
    return pl.pallas_call(
        _build_table_body,
        out_shape=jax.ShapeDtypeStruct((V_PAD, D_PROJ), jnp.float32),
    )(oct_lut_p, chr_lut_p, oct_tab_p, chr_tab_p, w_proj)


@functools.partial(
    pl.kernel,
    out_type=jax.ShapeDtypeStruct((B_TOTAL * D_PROJ,), jnp.float32),
    mesh=plsc.VectorSubcoreMesh(core_axis_name="c", subcore_axis_name="s"),
    compiler_params=pltpu.CompilerParams(needs_layout_passes=False),
    scratch_types=[
        pltpu.VMEM((V_PAD * D_PROJ,), jnp.float32),
        pltpu.VMEM((B_PER_W,), jnp.int32),
        pltpu.VMEM((CHUNK * D_PROJ,), jnp.float32),
        pltpu.VMEM((CHUNK * D_PROJ,), jnp.float32),
        pltpu.SemaphoreType.DMA,
        pltpu.SemaphoreType.DMA,
    ],
)
def _sc_gather(tok_hbm, f_hbm, out_hbm, f_v, tok_v, rows0, rows1, s0, s1):
    wid = lax.axis_index("s") * NC + lax.axis_index("c")
    base = wid * B_PER_W
    pltpu.sync_copy(f_hbm, f_v)
    pltpu.sync_copy(tok_hbm.at[pl.ds(base, B_PER_W)], tok_v)

    rows = (rows0, rows1)
    ssem = (s0, s1)

    lane = lax.iota(jnp.int32, L)
    col0 = [jnp.full((L,), cb * L, jnp.int32) + lane for cb in range(ROW_VECS)]

    def assemble(i, buf):
        @plsc.parallel_loop(0, CHUNK)
        def _(j):
            tok = plsc.load_gather(tok_v, [jnp.full((L,), i * CHUNK + j,
                                                    jnp.int32)])
            src = tok * D_PROJ
            dst = jnp.full((L,), j * D_PROJ, jnp.int32)
            vals = [plsc.load_gather(f_v, [src + col0[cb]])
                    for cb in range(ROW_VECS)]
            for cb in range(ROW_VECS):
                plsc.store_scatter(buf, [dst + col0[cb]], vals[cb])

    def out_slc(i):
        return out_hbm.at[pl.ds((base + i * CHUNK) * D_PROJ, CHUNK * D_PROJ)]

    @pl.loop(0, N_CHUNKS, step=2)
    def _(i0):
        for b in range(2):
            i = i0 + b
            @pl.when(i >= 2)
            def _():
                pltpu.make_async_copy(rows[b], out_slc(i - 2), ssem[b]).wait()
            assemble(i, rows[b])
            pltpu.async_copy(rows[b], out_slc(i), ssem[b])

    pltpu.make_async_copy(rows[0], out_slc(N_CHUNKS - 2), ssem[0]).wait()
    pltpu.make_async_copy(rows[1], out_slc(N_CHUNKS - 1), ssem[1]).wait()


def kernel(inp_tokens, octave_table, chroma_table, W_proj, oct_lut, chr_lut):
    f = _build_table(oct_lut, chr_lut, octave_table, chroma_table, W_proj)
    toks = inp_tokens.reshape(-1)
    out = _sc_gather(toks, f.reshape(-1))
    return out.reshape(inp_tokens.shape[0], inp_tokens.shape[1], D_PROJ)


# speedup vs baseline: 1.2715x; 1.2715x over previous
import functools

import jax
import jax.numpy as jnp
from jax import lax
from jax.experimental import pallas as pl
from jax.experimental.pallas import tpu as pltpu
from jax.experimental.pallas import tpu_sc as plsc

N_OCT = 8
N_CHR = 12
D_HALF = 128
D_PROJ = 512
V_PAD = 128
SCALE = float(D_PROJ ** 0.5)

NC = 2
NS = 16
NW = NC * NS
L = 16

B_TOTAL = 1024 * 200
TC_BLOCK = 512
B_TC = 102400
B_SC = B_TOTAL - B_TC
B_PER_W = B_SC // NW


def _build_table_body(oct_lut_ref, chr_lut_ref, oct_tab_ref, chr_tab_ref,
                      w_ref, f_ref):
    oct_ids = oct_lut_ref[...]
    chr_ids = chr_lut_ref[...]
    iota16 = lax.broadcasted_iota(jnp.int32, (V_PAD, 16), 1)
    oh_oct = (oct_ids == iota16).astype(jnp.float32)
    oh_chr = (chr_ids == iota16).astype(jnp.float32)
    emb_oct = jnp.dot(oh_oct, oct_tab_ref[...],
                      preferred_element_type=jnp.float32)
    emb_chr = jnp.dot(oh_chr, chr_tab_ref[...],
                      preferred_element_type=jnp.float32)
    emb = jnp.concatenate([emb_oct, emb_chr], axis=1)
    f_ref[...] = jnp.dot(emb, w_ref[...],
                         preferred_element_type=jnp.float32) * SCALE


def _build_table(oct_lut, chr_lut, octave_table, chroma_table, w_proj):
    oct_lut_p = jnp.concatenate(
        [oct_lut, jnp.full((V_PAD - oct_lut.shape[0],), N_OCT, jnp.int32)]
    ).reshape(V_PAD, 1)
    chr_lut_p = jnp.concatenate(
        [chr_lut, jnp.full((V_PAD - chr_lut.shape[0],), N_CHR, jnp.int32)]
    ).reshape(V_PAD, 1)
    oct_tab_p = jnp.zeros((16, D_HALF), jnp.float32).at[:N_OCT + 1].set(octave_table)
    chr_tab_p = jnp.zeros((16, D_HALF), jnp.float32).at[:N_CHR + 1].set(chroma_table)
    return pl.pallas_call(
        _build_table_body,
        out_shape=jax.ShapeDtypeStruct((V_PAD, D_PROJ), jnp.float32),
    )(oct_lut_p, chr_lut_p, oct_tab_p, chr_tab_p, w_proj)


def _tc_expand_body(tok_ref, f_ref, out_ref):
    toks = tok_ref[0, 0, :]
    vpad_iota = lax.broadcasted_iota(jnp.int32, (V_PAD, TC_BLOCK), 0)
    oh_t = (vpad_iota == toks[None, :]).astype(jnp.float32)
    out_ref[...] = lax.dot_general(
        oh_t, f_ref[...], (((0,), (0,)), ((), ())),
        preferred_element_type=jnp.float32)


def _tc_expand(toks_tc, f):
    nb = B_TC // TC_BLOCK
    return pl.pallas_call(
        _tc_expand_body,
        grid=(nb,),
        in_specs=[
            pl.BlockSpec((1, 1, TC_BLOCK), lambda i: (i, 0, 0)),
            pl.BlockSpec((V_PAD, D_PROJ), lambda i: (0, 0)),
        ],
        out_specs=pl.BlockSpec((TC_BLOCK, D_PROJ), lambda i: (i, 0)),
        out_shape=jax.ShapeDtypeStruct((B_TC, D_PROJ), jnp.float32),
    )(toks_tc.reshape(nb, 1, TC_BLOCK), f)


@functools.partial(
    pl.kernel,
    out_type=jax.ShapeDtypeStruct((B_SC * D_PROJ,), jnp.float32),
    mesh=plsc.VectorSubcoreMesh(core_axis_name="c", subcore_axis_name="s"),
    compiler_params=pltpu.CompilerParams(needs_layout_passes=False),
    scratch_types=[
        pltpu.VMEM((V_PAD * D_PROJ,), jnp.float32),
        pltpu.VMEM((B_PER_W,), jnp.int32),
        pltpu.SemaphoreType.DMA,
    ],
)
def _sc_gather(tok_hbm, f_hbm, out_hbm, f_v, tok_v, sem):
    wid = lax.axis_index("s") * NC + lax.axis_index("c")
    base = wid * B_PER_W
    pltpu.sync_copy(f_hbm, f_v)
    pltpu.sync_copy(tok_hbm.at[pl.ds(base, B_PER_W)], tok_v)

    @pl.loop(0, B_PER_W // L)
    def _(g):
        tvec = tok_v[pl.ds(g * L, L)]
        for l in range(L):
            t = tvec[l]
            pltpu.async_copy(
                f_v.at[pl.ds(t * D_PROJ, D_PROJ)],
                out_hbm.at[pl.ds((base + g * L + l) * D_PROJ, D_PROJ)],
                sem,
            )

    @pl.loop(0, B_PER_W)
    def _(j):
        pltpu.make_async_copy(
            f_v.at[pl.ds(0, D_PROJ)],
            out_hbm.at[pl.ds((base + j) * D_PROJ, D_PROJ)],
            sem,
        ).wait()


def kernel(inp_tokens, octave_table, chroma_table, W_proj, oct_lut, chr_lut):
    f = _build_table(oct_lut, chr_lut, octave_table, chroma_table, W_proj)
    toks = inp_tokens.reshape(-1)
    out_sc = _sc_gather(toks[B_TC:], f.reshape(-1)).reshape(B_SC, D_PROJ)
    out_tc = _tc_expand(toks[:B_TC], f)
    out = jnp.concatenate([out_tc, out_sc], axis=0)
    return out.reshape(inp_tokens.shape[0], inp_tokens.shape[1], D_PROJ)


# speedup vs baseline: 1.4924x; 1.1737x over previous
import functools

import jax
import jax.numpy as jnp
from jax import lax
from jax.experimental import pallas as pl
from jax.experimental.pallas import tpu as pltpu
from jax.experimental.pallas import tpu_sc as plsc

N_OCT = 8
N_CHR = 12
D_HALF = 128
D_PROJ = 512
V_PAD = 128
SCALE = float(D_PROJ ** 0.5)

NC = 2
NS = 16
NW = NC * NS
L = 16

B_TOTAL = 1024 * 200
B_PER_W = B_TOTAL // NW


def _build_table_body(oct_lut_ref, chr_lut_ref, oct_tab_ref, chr_tab_ref,
                      w_ref, f_ref):
    oct_ids = oct_lut_ref[...]
    chr_ids = chr_lut_ref[...]
    iota16 = lax.broadcasted_iota(jnp.int32, (V_PAD, 16), 1)
    oh_oct = (oct_ids == iota16).astype(jnp.float32)
    oh_chr = (chr_ids == iota16).astype(jnp.float32)
    emb_oct = jnp.dot(oh_oct, oct_tab_ref[...],
                      preferred_element_type=jnp.float32)
    emb_chr = jnp.dot(oh_chr, chr_tab_ref[...],
                      preferred_element_type=jnp.float32)
    emb = jnp.concatenate([emb_oct, emb_chr], axis=1)
    f_ref[...] = jnp.dot(emb, w_ref[...],
                         preferred_element_type=jnp.float32) * SCALE


def _build_table(oct_lut, chr_lut, octave_table, chroma_table, w_proj):
    oct_lut_p = jnp.concatenate(
        [oct_lut, jnp.full((V_PAD - oct_lut.shape[0],), N_OCT, jnp.int32)]
    ).reshape(V_PAD, 1)
    chr_lut_p = jnp.concatenate(
        [chr_lut, jnp.full((V_PAD - chr_lut.shape[0],), N_CHR, jnp.int32)]
    ).reshape(V_PAD, 1)
    oct_tab_p = jnp.zeros((16, D_HALF), jnp.float32).at[:N_OCT + 1].set(octave_table)
    chr_tab_p = jnp.zeros((16, D_HALF), jnp.float32).at[:N_CHR + 1].set(chroma_table)
    return pl.pallas_call(
        _build_table_body,
        out_shape=jax.ShapeDtypeStruct((V_PAD, D_PROJ), jnp.float32),
    )(oct_lut_p, chr_lut_p, oct_tab_p, chr_tab_p, w_proj)


@functools.partial(
    pl.kernel,
    out_type=jax.ShapeDtypeStruct((B_TOTAL * D_PROJ,), jnp.float32),
    mesh=plsc.VectorSubcoreMesh(core_axis_name="c", subcore_axis_name="s"),
    compiler_params=pltpu.CompilerParams(needs_layout_passes=False),
    scratch_types=[
        pltpu.VMEM((V_PAD * D_PROJ,), jnp.float32),
        pltpu.VMEM((B_PER_W,), jnp.int32),
        pltpu.SemaphoreType.DMA,
    ],
)
def _sc_gather(tok_hbm, f_hbm, out_hbm, f_v, tok_v, sem):
    wid = lax.axis_index("s") * NC + lax.axis_index("c")
    base = wid * B_PER_W
    pltpu.sync_copy(f_hbm, f_v)
    pltpu.sync_copy(tok_hbm.at[pl.ds(base, B_PER_W)], tok_v)

    @pl.loop(0, B_PER_W // L)
    def _(g):
        tvec = tok_v[pl.ds(g * L, L)]
        for l in range(L):
            t = tvec[l]
            pltpu.async_copy(
                f_v.at[pl.ds(t * D_PROJ, D_PROJ)],
                out_hbm.at[pl.ds((base + g * L + l) * D_PROJ, D_PROJ)],
                sem,
            )

    @pl.loop(0, B_PER_W // 64)
    def _(d):
        blk = out_hbm.at[pl.ds((base + d * 64) * D_PROJ, 64 * D_PROJ)]
        pltpu.make_async_copy(f_v.at[pl.ds(0, 64 * D_PROJ)], blk, sem).wait()


def kernel(inp_tokens, octave_table, chroma_table, W_proj, oct_lut, chr_lut):
    f = _build_table(oct_lut, chr_lut, octave_table, chroma_table, W_proj)
    toks = inp_tokens.reshape(-1)
    out = _sc_gather(toks, f.reshape(-1))
    return out.reshape(inp_tokens.shape[0], inp_tokens.shape[1], D_PROJ)
